# baseline (speedup 1.0000x reference)
import functools

import jax
import jax.numpy as jnp
from jax import lax
from jax.experimental import pallas as pl
from jax.experimental.pallas import tpu as pltpu
from jax.experimental.pallas import tpu_sc as plsc

N = 10000
E = 320000
D = 128
LAYERS = 5

NC = 2
NS = 16
NW = NC * NS

N_P = 10240
TRASH = N

CHUNK = 128
K = 80
E_P = NW * K * CHUNK

CL = 16

ROWS_PW = N_P // NS

XCHUNK = 80
XK = 4

_MESH = plsc.VectorSubcoreMesh(core_axis_name="c", subcore_axis_name="s")


@functools.partial(
    pl.kernel,
    out_type=jax.ShapeDtypeStruct((N_P, D), jnp.float32),
    mesh=_MESH,
    scratch_types=[
        pltpu.VMEM((XK, XCHUNK), jnp.int32),
        pltpu.VMEM((XCHUNK, D), jnp.float32),
        pltpu.SemaphoreType.DMA,
    ],
)
def _emb_gather(emb_hbm, xp_hbm, out_hbm, idx_v, rows_v, sem):
    c = lax.axis_index("c")
    s = lax.axis_index("s")
    wid = s * NC + c
    pltpu.sync_copy(xp_hbm.at[wid], idx_v)
    base = wid * (XK * XCHUNK)

    def body(j, carry):
        pltpu.async_copy(emb_hbm.at[idx_v.at[j]], rows_v, sem).wait()
        pltpu.sync_copy(rows_v, out_hbm.at[pl.ds(base + j * XCHUNK, XCHUNK)])
        return carry

    lax.fori_loop(0, XK, body, 0)


@functools.partial(
    pl.kernel,
    out_type=jax.ShapeDtypeStruct((NC, N_P, D), jnp.float32),
    mesh=_MESH,
    scratch_types=[
        pltpu.VMEM((K, CHUNK), jnp.int32),
        pltpu.VMEM((K, CHUNK), jnp.int32),
        pltpu.VMEM((CHUNK, D), jnp.float32),
        pltpu.VMEM((CHUNK // 2, D), jnp.float32),
        pltpu.VMEM_SHARED((N_P, D), jnp.float32),
        pltpu.SemaphoreType.DMA,
        pltpu.SemaphoreType.DMA,
    ],
)
def _agg_only(h_hbm, srcp_hbm, dstp_hbm, out_agg,
              src_v, dst_v, buf_a, buf_b, agg_sh, sem_a, sem_b):
    c = lax.axis_index("c")
    s = lax.axis_index("s")
    wid = s * NC + c

    pltpu.sync_copy(srcp_hbm.at[wid], src_v)
    pltpu.sync_copy(dstp_hbm.at[wid], dst_v)

    z16 = jnp.zeros((16,), jnp.float32)

    def zrow(i, carry):
        def zcol(k, carry2):
            buf_a[i, pl.ds(k * 16, 16)] = z16
            return carry2
        lax.fori_loop(0, D // 16, zcol, 0)
        return carry

    lax.fori_loop(0, CHUNK, zrow, 0)
    for t in range(ROWS_PW // CHUNK):
        pltpu.sync_copy(
            buf_a, agg_sh.at[pl.ds(s * ROWS_PW + t * CHUNK, CHUNK)])
    plsc.subcore_barrier()

    H = CHUNK // 2

    def body(j, carry):
        pltpu.async_copy(h_hbm.at[src_v.at[j, pl.ds(H, H)]], buf_b, sem_b)
        pltpu.async_copy(h_hbm.at[src_v.at[j, pl.ds(0, H)]],
                         buf_a.at[pl.ds(0, H)], sem_a).wait()
        pltpu.sync_copy(buf_a.at[pl.ds(0, H)],
                        agg_sh.at[dst_v.at[j, pl.ds(0, H)]], add=True)
        pltpu.make_async_copy(h_hbm.at[src_v.at[j, pl.ds(H, H)]], buf_b,
                              sem_b).wait()
        pltpu.sync_copy(buf_b, agg_sh.at[dst_v.at[j, pl.ds(H, H)]], add=True)
        return carry

    lax.fori_loop(0, K, body, 0)
    plsc.subcore_barrier()

    pltpu.sync_copy(agg_sh.at[pl.ds(s * ROWS_PW, ROWS_PW)],
                    out_agg.at[c, pl.ds(s * ROWS_PW, ROWS_PW)])


def _tc_body(agg0, agg1, cnt0, cnt1, h, wl, wr, bb, out):
    cnt = cnt0[:, 0:1] + cnt1[:, 0:1]
    inv = 1.0 / jnp.maximum(cnt, 1.0)
    mean = (agg0[...] + agg1[...]) * inv
    acc = jnp.dot(mean, wl[...], preferred_element_type=jnp.float32)
    acc = acc + jnp.dot(h[...], wr[...], preferred_element_type=jnp.float32)
    out[...] = jnp.maximum(acc + bb[...], 0.0)


_BM = 512


def _tc_layer(agg0, agg1, cnt0, cnt1, h, wl, wr, bb):
    grid = (N_P // _BM,)
    row = lambda i: (i, 0)
    fixed = lambda i: (0, 0)
    return pl.pallas_call(
        _tc_body,
        grid=grid,
        in_specs=[
            pl.BlockSpec((_BM, D), row),
            pl.BlockSpec((_BM, D), row),
            pl.BlockSpec((_BM, D), row),
            pl.BlockSpec((_BM, D), row),
            pl.BlockSpec((_BM, D), row),
            pl.BlockSpec((D, D), fixed),
            pl.BlockSpec((D, D), fixed),
            pl.BlockSpec((1, D), fixed),
        ],
        out_specs=pl.BlockSpec((_BM, D), row),
        out_shape=jax.ShapeDtypeStruct((N_P, D), jnp.float32),
    )(agg0, agg1, cnt0, cnt1, h, wl, wr, bb)


def kernel(x, edges, emb, Wl, Wr, b):
    src = edges[0].astype(jnp.int32)
    dst = edges[1].astype(jnp.int32)
    srcp = jnp.concatenate(
        [src, jnp.zeros((E_P - E,), jnp.int32)]).reshape(NW, K, CHUNK)
    dstp = jnp.concatenate(
        [dst, jnp.full((E_P - E,), TRASH, jnp.int32)]).reshape(NW, K, CHUNK)
    xp = jnp.concatenate(
        [x.astype(jnp.int32), jnp.zeros((N_P - N,), jnp.int32)]
    ).reshape(NW, XK, XCHUNK)

    ones_mat = jnp.ones((N_P, D), jnp.float32)
    bb = b.reshape(LAYERS, 1, D)

    h = _emb_gather(emb, xp)
    cnt = _agg_only(ones_mat, srcp, dstp)
    cnt0, cnt1 = cnt[0], cnt[1]
    for i in range(LAYERS):
        agg = _agg_only(h, srcp, dstp)
        h = _tc_layer(agg[0], agg[1], cnt0, cnt1, h, Wl[i], Wr[i], bb[i])
    return h[:N]


# speedup vs baseline: 1.0110x; 1.0110x over previous
import functools

import jax
import jax.numpy as jnp
from jax import lax
from jax.experimental import pallas as pl
from jax.experimental.pallas import tpu as pltpu
from jax.experimental.pallas import tpu_sc as plsc

N = 10000
E = 320000
D = 128
LAYERS = 5

NC = 2
NS = 16
NW = NC * NS

N_P = 10240
TRASH = N

CHUNK = 128
K = 80
E_P = NW * K * CHUNK

CL = 16

ROWS_PW = N_P // NS

XCHUNK = 80
XK = 4

_MESH = plsc.VectorSubcoreMesh(core_axis_name="c", subcore_axis_name="s")


@functools.partial(
    pl.kernel,
    out_type=jax.ShapeDtypeStruct((N_P, D), jnp.float32),
    mesh=_MESH,
    scratch_types=[
        pltpu.VMEM((XK, XCHUNK), jnp.int32),
        pltpu.VMEM((XCHUNK, D), jnp.float32),
        pltpu.SemaphoreType.DMA,
    ],
)
def _emb_gather(emb_hbm, xp_hbm, out_hbm, idx_v, rows_v, sem):
    c = lax.axis_index("c")
    s = lax.axis_index("s")
    wid = s * NC + c
    pltpu.sync_copy(xp_hbm.at[wid], idx_v)
    base = wid * (XK * XCHUNK)

    def body(j, carry):
        pltpu.async_copy(emb_hbm.at[idx_v.at[j]], rows_v, sem).wait()
        pltpu.sync_copy(rows_v, out_hbm.at[pl.ds(base + j * XCHUNK, XCHUNK)])
        return carry

    lax.fori_loop(0, XK, body, 0)


@functools.partial(
    pl.kernel,
    out_type=jax.ShapeDtypeStruct((NC, N_P, D), jnp.float32),
    mesh=_MESH,
    scratch_types=[
        pltpu.VMEM((K, CHUNK), jnp.int32),
        pltpu.VMEM((K, CHUNK), jnp.int32),
        pltpu.VMEM((CHUNK, D), jnp.float32),
        pltpu.VMEM((CHUNK, D), jnp.float32),
        pltpu.VMEM_SHARED((N_P, D), jnp.float32),
        pltpu.SemaphoreType.DMA,
        pltpu.SemaphoreType.DMA,
    ],
)
def _agg_only(h_hbm, srcp_hbm, dstp_hbm, out_agg,
              src_v, dst_v, buf_a, buf_b, agg_sh, sem_a, sem_b):
    c = lax.axis_index("c")
    s = lax.axis_index("s")
    wid = s * NC + c

    pltpu.sync_copy(srcp_hbm.at[wid], src_v)
    pltpu.sync_copy(dstp_hbm.at[wid], dst_v)

    z16 = jnp.zeros((16,), jnp.float32)

    def zrow(i, carry):
        def zcol(k, carry2):
            buf_a[i, pl.ds(k * 16, 16)] = z16
            return carry2
        lax.fori_loop(0, D // 16, zcol, 0)
        return carry

    lax.fori_loop(0, CHUNK, zrow, 0)
    for t in range(ROWS_PW // CHUNK):
        pltpu.sync_copy(
            buf_a, agg_sh.at[pl.ds(s * ROWS_PW + t * CHUNK, CHUNK)])
    plsc.subcore_barrier()

    def body(j, carry):
        pltpu.async_copy(h_hbm.at[src_v.at[j]], buf_a, sem_a).wait()
        pltpu.sync_copy(buf_a, agg_sh.at[dst_v.at[j]], add=True)
        return carry

    lax.fori_loop(0, K, body, 0)
    plsc.subcore_barrier()

    for t in range(ROWS_PW // CHUNK):
        r0 = s * ROWS_PW + t * CHUNK
        pltpu.sync_copy(agg_sh.at[pl.ds(r0, CHUNK)],
                        out_agg.at[c, pl.ds(r0, CHUNK)])


def _tc_body(agg0, agg1, cnt0, cnt1, h, wl, wr, bb, out):
    cnt = cnt0[:, 0:1] + cnt1[:, 0:1]
    inv = 1.0 / jnp.maximum(cnt, 1.0)
    mean = (agg0[...] + agg1[...]) * inv
    acc = jnp.dot(mean, wl[...], preferred_element_type=jnp.float32)
    acc = acc + jnp.dot(h[...], wr[...], preferred_element_type=jnp.float32)
    out[...] = jnp.maximum(acc + bb[...], 0.0)


_BM = 512


def _tc_layer(agg0, agg1, cnt0, cnt1, h, wl, wr, bb):
    grid = (N_P // _BM,)
    row = lambda i: (i, 0)
    fixed = lambda i: (0, 0)
    return pl.pallas_call(
        _tc_body,
        grid=grid,
        in_specs=[
            pl.BlockSpec((_BM, D), row),
            pl.BlockSpec((_BM, D), row),
            pl.BlockSpec((_BM, D), row),
            pl.BlockSpec((_BM, D), row),
            pl.BlockSpec((_BM, D), row),
            pl.BlockSpec((D, D), fixed),
            pl.BlockSpec((D, D), fixed),
            pl.BlockSpec((1, D), fixed),
        ],
        out_specs=pl.BlockSpec((_BM, D), row),
        out_shape=jax.ShapeDtypeStruct((N_P, D), jnp.float32),
    )(agg0, agg1, cnt0, cnt1, h, wl, wr, bb)


def kernel(x, edges, emb, Wl, Wr, b):
    src = edges[0].astype(jnp.int32)
    dst = edges[1].astype(jnp.int32)
    srcp = jnp.concatenate(
        [src, jnp.zeros((E_P - E,), jnp.int32)]).reshape(NW, K, CHUNK)
    dstp = jnp.concatenate(
        [dst, jnp.full((E_P - E,), TRASH, jnp.int32)]).reshape(NW, K, CHUNK)
    xp = jnp.concatenate(
        [x.astype(jnp.int32), jnp.zeros((N_P - N,), jnp.int32)]
    ).reshape(NW, XK, XCHUNK)

    ones_mat = jnp.ones((N_P, D), jnp.float32)
    bb = b.reshape(LAYERS, 1, D)

    h = _emb_gather(emb, xp)
    cnt = _agg_only(ones_mat, srcp, dstp)
    cnt0, cnt1 = cnt[0], cnt[1]
    for i in range(LAYERS):
        agg = _agg_only(h, srcp, dstp)
        h = _tc_layer(agg[0], agg[1], cnt0, cnt1, h, Wl[i], Wr[i], bb[i])
    return h[:N]


# speedup vs baseline: 1.1939x; 1.1809x over previous
import functools

import jax
import jax.numpy as jnp
from jax import lax
from jax.experimental import pallas as pl
from jax.experimental.pallas import tpu as pltpu
from jax.experimental.pallas import tpu_sc as plsc

N = 10000
E = 320000
D = 128
LAYERS = 5

NC = 2
NS = 16
NW = NC * NS

N_P = 10240
TRASH = N

CHUNK = 128
K = 80
E_P = NW * K * CHUNK

CL = 16

ROWS_PW = N_P // NS

XCHUNK = 80
XK = 4

_MESH = plsc.VectorSubcoreMesh(core_axis_name="c", subcore_axis_name="s")


@functools.partial(
    pl.kernel,
    out_type=jax.ShapeDtypeStruct((N_P, D), jnp.float32),
    mesh=_MESH,
    scratch_types=[
        pltpu.VMEM((XK, XCHUNK), jnp.int32),
        pltpu.VMEM((XCHUNK, D), jnp.float32),
        pltpu.SemaphoreType.DMA,
    ],
)
def _emb_gather(emb_hbm, xp_hbm, out_hbm, idx_v, rows_v, sem):
    c = lax.axis_index("c")
    s = lax.axis_index("s")
    wid = s * NC + c
    pltpu.sync_copy(xp_hbm.at[wid], idx_v)
    base = wid * (XK * XCHUNK)

    def body(j, carry):
        pltpu.async_copy(emb_hbm.at[idx_v.at[j]], rows_v, sem).wait()
        pltpu.sync_copy(rows_v, out_hbm.at[pl.ds(base + j * XCHUNK, XCHUNK)])
        return carry

    lax.fori_loop(0, XK, body, 0)


@functools.partial(
    pl.kernel,
    out_type=jax.ShapeDtypeStruct((NC, N_P, D), jnp.float32),
    mesh=_MESH,
    scratch_types=[
        pltpu.VMEM((K, CHUNK), jnp.int32),
        pltpu.VMEM((K, CHUNK), jnp.int32),
        pltpu.VMEM((CHUNK, D), jnp.float32),
        pltpu.VMEM_SHARED((N_P, D), jnp.float32),
        pltpu.SemaphoreType.DMA,
    ],
)
def _agg_only(h_hbm, srcp_hbm, dstp_hbm, z_hbm, out_agg,
              src_v, dst_v, buf, agg_sh, sem):
    c = lax.axis_index("c")
    s = lax.axis_index("s")
    wid = s * NC + c

    pltpu.sync_copy(srcp_hbm.at[wid], src_v)
    pltpu.sync_copy(dstp_hbm.at[wid], dst_v)

    pltpu.sync_copy(z_hbm, agg_sh.at[pl.ds(s * ROWS_PW, ROWS_PW)])
    plsc.subcore_barrier()

    def body(j, carry):
        pltpu.async_copy(h_hbm.at[src_v.at[j]], buf, sem).wait()
        pltpu.sync_copy(buf, agg_sh.at[dst_v.at[j]], add=True)
        return carry

    lax.fori_loop(0, K, body, 0)
    plsc.subcore_barrier()

    pltpu.sync_copy(agg_sh.at[pl.ds(s * ROWS_PW, ROWS_PW)],
                    out_agg.at[c, pl.ds(s * ROWS_PW, ROWS_PW)])


@functools.partial(
    pl.kernel,
    out_type=jax.ShapeDtypeStruct((NC, N_P, D), jnp.float32),
    mesh=_MESH,
    scratch_types=[
        pltpu.VMEM((K, CHUNK), jnp.int32),
        pltpu.VMEM((CHUNK, D), jnp.float32),
        pltpu.VMEM_SHARED((N_P, D), jnp.float32),
    ],
)
def _deg_count(dstp_hbm, zc_hbm, ones_hbm, out_cnt, dst_v, ones_v, cnt_sh):
    c = lax.axis_index("c")
    s = lax.axis_index("s")
    wid = s * NC + c
    pltpu.sync_copy(dstp_hbm.at[wid], dst_v)
    pltpu.sync_copy(ones_hbm, ones_v)
    pltpu.sync_copy(zc_hbm, cnt_sh.at[pl.ds(s * ROWS_PW, ROWS_PW)])
    plsc.subcore_barrier()

    def body(j, carry):
        pltpu.sync_copy(ones_v, cnt_sh.at[dst_v.at[j]], add=True)
        return carry

    lax.fori_loop(0, K, body, 0)
    plsc.subcore_barrier()
    pltpu.sync_copy(cnt_sh.at[pl.ds(s * ROWS_PW, ROWS_PW)],
                    out_cnt.at[c, pl.ds(s * ROWS_PW, ROWS_PW)])


def _tc_body(agg0, agg1, cnt0, cnt1, h, wl, wr, bb, out):
    cnt = cnt0[:, 0:1] + cnt1[:, 0:1]
    inv = 1.0 / jnp.maximum(cnt, 1.0)
    mean = (agg0[...] + agg1[...]) * inv
    acc = jnp.dot(mean, wl[...], preferred_element_type=jnp.float32)
    acc = acc + jnp.dot(h[...], wr[...], preferred_element_type=jnp.float32)
    out[...] = jnp.maximum(acc + bb[...], 0.0)


_BM = 512


def _tc_layer(agg0, agg1, cnt0, cnt1, h, wl, wr, bb):
    grid = (N_P // _BM,)
    row = lambda i: (i, 0)
    fixed = lambda i: (0, 0)
    return pl.pallas_call(
        _tc_body,
        grid=grid,
        in_specs=[
            pl.BlockSpec((_BM, D), row),
            pl.BlockSpec((_BM, D), row),
            pl.BlockSpec((_BM, D), row),
            pl.BlockSpec((_BM, D), row),
            pl.BlockSpec((_BM, D), row),
            pl.BlockSpec((D, D), fixed),
            pl.BlockSpec((D, D), fixed),
            pl.BlockSpec((1, D), fixed),
        ],
        out_specs=pl.BlockSpec((_BM, D), row),
        out_shape=jax.ShapeDtypeStruct((N_P, D), jnp.float32),
    )(agg0, agg1, cnt0, cnt1, h, wl, wr, bb)


def kernel(x, edges, emb, Wl, Wr, b):
    src = edges[0].astype(jnp.int32)
    dst = edges[1].astype(jnp.int32)
    srcp = jnp.concatenate(
        [src, jnp.zeros((E_P - E,), jnp.int32)]).reshape(NW, K, CHUNK)
    dstp = jnp.concatenate(
        [dst, jnp.full((E_P - E,), TRASH, jnp.int32)]).reshape(NW, K, CHUNK)
    xp = jnp.concatenate(
        [x.astype(jnp.int32), jnp.zeros((N_P - N,), jnp.int32)]
    ).reshape(NW, XK, XCHUNK)

    z = jnp.zeros((ROWS_PW, D), jnp.float32)
    ones = jnp.ones((CHUNK, D), jnp.float32)
    bb = b.reshape(LAYERS, 1, D)

    h = _emb_gather(emb, xp)
    cnt = _deg_count(dstp, z, ones)
    cnt0, cnt1 = cnt[0], cnt[1]
    for i in range(LAYERS):
        agg = _agg_only(h, srcp, dstp, z)
        h = _tc_layer(agg[0], agg[1], cnt0, cnt1, h, Wl[i], Wr[i], bb[i])
    return h[:N]
